# baseline (speedup 1.0000x reference)
import functools

import jax
import jax.numpy as jnp
from jax import lax
from jax.experimental import pallas as pl
from jax.experimental.pallas import tpu as pltpu
from jax.experimental.pallas import tpu_sc as plsc

V = 10000
D = 128
B = 320000

_info = plsc.get_sparse_core_info()
NC, NS = _info.num_cores, _info.num_subcores
NW = NC * NS
B_PER_W = B // NW
C = 80
R = 2 * C
S = B_PER_W // R
G = S // 2

_mesh = plsc.VectorSubcoreMesh(core_axis_name="c", subcore_axis_name="s")


@functools.partial(
    pl.kernel,
    mesh=_mesh,
    out_type=jax.ShapeDtypeStruct((B, D), jnp.float32),
    scratch_types=[
        pltpu.VMEM((R,), jnp.int32),
        pltpu.VMEM((R,), jnp.int32),
        pltpu.VMEM((2, R, D), jnp.float32),
        pltpu.VMEM_SHARED((V, D), jnp.float32),
        pltpu.SemaphoreType.DMA((2,)),
        pltpu.SemaphoreType.DMA((2,)),
        pltpu.SemaphoreType.DMA((2,)),
    ],
)
def _gather_sc(x_hbm, idx_hbm, out_hbm, idx_v0, idx_v1, rows_v, xs, isem, gsem,
               ssem):
    idx_bufs = (idx_v0, idx_v1)
    sid = lax.axis_index("s")
    wid = sid * NC + lax.axis_index("c")
    base_w = wid * B_PER_W

    RS = 632
    @pl.when(sid < NS - 1)
    def _():
        r0 = pl.multiple_of(sid * RS, 8)
        pltpu.sync_copy(x_hbm.at[pl.ds(r0, RS)], xs.at[pl.ds(r0, RS)])

    @pl.when(sid == NS - 1)
    def _():
        r0 = (NS - 1) * RS
        pltpu.sync_copy(x_hbm.at[pl.ds(r0, V - r0)], xs.at[pl.ds(r0, V - r0)])

    def idx_copy(s, b, n=R):
        off = pl.multiple_of(B + base_w + s * R, 8)
        return pltpu.make_async_copy(
            idx_hbm.at[pl.ds(off, n)], idx_bufs[b].at[pl.ds(0, n)], isem.at[b])

    def gather_copy(b, h):
        return pltpu.make_async_copy(
            xs.at[idx_bufs[b].at[pl.ds(h * C, C)]],
            rows_v.at[b, pl.ds(h * C, C)], gsem.at[b])

    def gather_wait(b, n=R):
        pltpu.make_async_copy(
            xs.at[idx_bufs[b].at[pl.ds(0, C)]],
            rows_v.at[b, pl.ds(0, n)], gsem.at[b]).wait()

    def store_copy(s, b, n=R):
        off = pl.multiple_of(base_w + s * R, 8)
        return pltpu.make_async_copy(
            rows_v.at[b, pl.ds(0, n)], out_hbm.at[pl.ds(off, n)], ssem.at[b])

    idx_copy(0, 0).start()
    idx_copy(1, 1).start()
    plsc.subcore_barrier()

    def outer(g, carry):
        for b in range(2):
            s = 2 * g + b
            @pl.when(g > 0)
            def _():
                store_copy(0, b).wait()

            pb = 1 - b
            if b == 1:
                gather_wait(pb)
                store_copy(s - 1, pb).start()
                @pl.when(g < G - 1)
                def _():
                    idx_copy(s + 1, pb).start()
            else:
                @pl.when(g > 0)
                def _():
                    gather_wait(pb)
                    store_copy(s - 1, pb).start()
                    idx_copy(s + 1, pb).start()

            idx_copy(0, b).wait()
            gather_copy(b, 0).start()
            gather_copy(b, 1).start()
        return carry

    lax.fori_loop(0, G, outer, 0)

    gather_wait(1)
    store_copy(S - 1, 1).start()
    store_copy(0, 0).wait()
    idx_copy(S, 0, n=C).start()
    idx_copy(0, 0, n=C).wait()
    gather_copy(0, 0).start()
    gather_wait(0, n=C)
    store_copy(S, 0, n=C).start()
    store_copy(0, 1).wait()
    store_copy(0, 0, n=C).wait()


def kernel(x, edge_index):
    return _gather_sc(x, edge_index.reshape(-1))


# speedup vs baseline: 1.0319x; 1.0319x over previous
import functools

import jax
import jax.numpy as jnp
from jax import lax
from jax.experimental import pallas as pl
from jax.experimental.pallas import tpu as pltpu
from jax.experimental.pallas import tpu_sc as plsc

V = 10000
D = 128
B = 320000

_info = plsc.get_sparse_core_info()
NC, NS = _info.num_cores, _info.num_subcores
NW = NC * NS
B_PER_W = B // NW
C = 80
N_CHUNKS = B_PER_W // C
NBUF = 4
G = (N_CHUNKS - 1) // NBUF

_mesh = plsc.VectorSubcoreMesh(core_axis_name="c", subcore_axis_name="s")


@functools.partial(
    pl.kernel,
    mesh=_mesh,
    out_type=jax.ShapeDtypeStruct((B, D), jnp.float32),
    scratch_types=[
        pltpu.VMEM((NBUF, C), jnp.int32),
        pltpu.VMEM((NBUF, C, D), jnp.float32),
        pltpu.VMEM_SHARED((V, D), jnp.float32),
        pltpu.SemaphoreType.DMA((NBUF,)),
        pltpu.SemaphoreType.DMA((NBUF,)),
        pltpu.SemaphoreType.DMA((NBUF,)),
    ],
)
def _gather_sc(x_hbm, idx_hbm, out_hbm, idx_v, rows_v, xs, isem, gsem, ssem):
    sid = lax.axis_index("s")
    wid = sid * NC + lax.axis_index("c")
    base_w = wid * B_PER_W

    RS = 632
    @pl.when(sid < NS - 1)
    def _():
        r0 = pl.multiple_of(sid * RS, 8)
        pltpu.sync_copy(x_hbm.at[pl.ds(r0, RS)], xs.at[pl.ds(r0, RS)])

    @pl.when(sid == NS - 1)
    def _():
        r0 = (NS - 1) * RS
        pltpu.sync_copy(x_hbm.at[pl.ds(r0, V - r0)], xs.at[pl.ds(r0, V - r0)])

    def idx_copy(i, b):
        off = pl.multiple_of(B + base_w + i * C, 8)
        return pltpu.make_async_copy(
            idx_hbm.at[pl.ds(off, C)], idx_v.at[b], isem.at[b])

    def gather_copy(b):
        return pltpu.make_async_copy(
            xs.at[idx_v.at[b]], rows_v.at[b], gsem.at[b])

    def store_copy(i, b):
        off = pl.multiple_of(base_w + i * C, 8)
        return pltpu.make_async_copy(
            rows_v.at[b], out_hbm.at[pl.ds(off, C)], ssem.at[b])

    idx_copy(0, 0).start()
    idx_copy(1, 1).start()
    plsc.subcore_barrier()

    SKEW = 2

    def outer(g, carry):
        for b in range(NBUF):
            i = g * NBUF + b
            @pl.when(g > 0)
            def _():
                store_copy(0, b).wait()

            pb = (b - SKEW) % NBUF
            if b >= SKEW:
                gather_copy(pb).wait()
                store_copy(i - SKEW, pb).start()
            else:
                @pl.when(g > 0)
                def _():
                    gather_copy(pb).wait()
                    store_copy(g * NBUF + b - SKEW, pb).start()

            @pl.when(i <= N_CHUNKS - 1 - SKEW)
            def _():
                idx_copy(i + SKEW, (b + SKEW) % NBUF).start()

            idx_copy(0, b).wait()
            gather_copy(b).start()
        return carry

    lax.fori_loop(0, G, outer, 0)

    gather_copy(2).wait()
    store_copy(N_CHUNKS - 3, 2).start()
    store_copy(0, 0).wait()
    idx_copy(0, 0).wait()
    gather_copy(0).start()
    gather_copy(3).wait()
    store_copy(N_CHUNKS - 2, 3).start()
    gather_copy(0).wait()
    store_copy(N_CHUNKS - 1, 0).start()
    for b in range(1, NBUF):
        store_copy(0, b).wait()
    store_copy(0, 0).wait()


def kernel(x, edge_index):
    return _gather_sc(x, edge_index.reshape(-1))
